# baseline (speedup 1.0000x reference)
import functools

import jax
import jax.numpy as jnp
from jax import lax
from jax.experimental import pallas as pl
from jax.experimental.pallas import tpu as pltpu
from jax.experimental.pallas import tpu_sc as plsc

NC = 2
NSUB = 16
NW = NC * NSUB
NSLAB = 3
SLABW = 32
ECH = 256
MCH = 384

_SC_PARAMS = pltpu.CompilerParams(
    needs_layout_passes=False, use_tc_tiling_on_sc=False)


def _pick_block(n, target):
    for b in range(target, 0, -1):
        if n % b == 0:
            return b
    return n


def _pack_h(h):
    n, d = h.shape
    hp = jnp.concatenate(
        [h, jnp.zeros((n, SLABW * NSLAB - d), jnp.float32)], axis=-1)
    return jnp.stack(
        [hp[:, SLABW * s:SLABW * (s + 1)] for s in range(NSLAB)])


def _init_body(x_ref, win_ref, bin_ref, we1a_ref, we1b_ref, be1_ref,
               h5_ref, g1_ref, g2_ref):
    X = x_ref[...]
    H = jnp.tanh(X @ win_ref[...] + bin_ref[...])
    h = jnp.concatenate([H, X], axis=-1)
    h5_ref[...] = _pack_h(h)
    g1_ref[...] = (h @ we1a_ref[...]).astype(jnp.bfloat16)
    g2_ref[...] = (h @ we1b_ref[...] + be1_ref[...]).astype(jnp.bfloat16)


def _unpack(m5, d):
    parts = [m5[s] for s in range(NSLAB - 1)]
    parts.append(m5[NSLAB - 1][:, :d - SLABW * (NSLAB - 1)])
    return jnp.concatenate(parts, axis=-1)


def _node_body(m2_ref, h5_ref, x_ref, wn1_ref, bn1_ref, wn2_ref,
               bn2_ref, we1a_ref, we1b_ref, be1_ref,
               h5o_ref, g1_ref, g2_ref):
    X = x_ref[...]
    d = wn1_ref.shape[0] // 3
    mi = _unpack(m2_ref[0], d)
    mo = _unpack(m2_ref[1], d)
    h = _unpack(h5_ref[...], d)
    M = jnp.concatenate([mi, mo, h], axis=-1)
    T = jnp.tanh(M @ wn1_ref[...] + bn1_ref[...])
    Hn = jnp.tanh(T @ wn2_ref[...] + bn2_ref[...])
    hn = jnp.concatenate([Hn, X], axis=-1)
    h5o_ref[...] = _pack_h(hn)
    g1_ref[...] = (hn @ we1a_ref[...]).astype(jnp.bfloat16)
    g2_ref[...] = (hn @ we1b_ref[...] + be1_ref[...]).astype(jnp.bfloat16)


def _full(shape):
    return pl.BlockSpec(shape, lambda i: (0,) * len(shape))


def _epass_body(E, EPT, g1_hbm, g2_hbm, cp2_hbm, rp2_hbm, wtab_hbm,
                e_hbm, cidxA, ridxA, cidxB, ridxB, g1bA, g2bA, g1bB, g2bB,
                eb2, ebuf, wbuf, semA, semB):
    cid = lax.axis_index("c")
    sid = lax.axis_index("s")
    wid = sid * NC + cid
    pltpu.sync_copy(wtab_hbm, wbuf)
    w2 = [wbuf[j] for j in range(4)]
    c0v = wbuf[4]
    iota = lax.iota(jnp.int32, 16)
    nrow = ECH // 128
    nch = EPT // ECH

    def load(c, cidx, ridx, g1b, g2b, sem):
        rbase = wid * (EPT // 128) + c * nrow
        pltpu.sync_copy(cp2_hbm.at[pl.ds(rbase, nrow)], cidx)
        pltpu.sync_copy(rp2_hbm.at[pl.ds(rbase, nrow)], ridx)
        for j in range(nrow):
            pltpu.async_copy(
                g1_hbm.at[cidx.at[j]], g1b.at[pl.ds(j * 128, 128)], sem)
            pltpu.async_copy(
                g2_hbm.at[ridx.at[j]], g2b.at[pl.ds(j * 128, 128)], sem)

    def drain(g1b, g2b, sem):
        pltpu.make_async_copy(g1_hbm.at[pl.ds(0, ECH)], g1b, sem).wait()
        pltpu.make_async_copy(g2_hbm.at[pl.ds(0, ECH)], g2b, sem).wait()

    def compute(c, g1b, g2b):
        @plsc.parallel_loop(0, ECH, unroll=4)
        def edge(i):
            acc = jnp.zeros((16,), jnp.float32)
            for j in range(2):
                ua = g1b[i, pl.ds(j * 16, 16)]
                ub = g2b[i, pl.ds(j * 16, 16)]
                for p in range(2):
                    if p == 0:
                        fa = lax.bitcast_convert_type(
                            lax.shift_left(ua, 16), jnp.float32)
                        fb = lax.bitcast_convert_type(
                            lax.shift_left(ub, 16), jnp.float32)
                    else:
                        fa = lax.bitcast_convert_type(
                            ua & jnp.int32(-65536), jnp.float32)
                        fb = lax.bitcast_convert_type(
                            ub & jnp.int32(-65536), jnp.float32)
                    v = fa + fb
                    den = jnp.exp(v * (-2.0)) + 1.0
                    acc = acc + w2[2 * j + p] / den
            eb2[pl.ds(i * 16, 16)] = acc
        gbase = wid * EPT + c * ECH

        @plsc.parallel_loop(0, ECH // 16, unroll=2)
        def sig(g):
            base = g * 256 + iota * 16
            sv = jnp.zeros((16,), jnp.float32)
            for k in range(16):
                sv = sv + plsc.load_gather(eb2, [base + k])
            ev = 1.0 / (1.0 + jnp.exp(-(sv + c0v)))
            pos = gbase + g * 16 + iota
            ebuf[pl.ds(g * 16, 16)] = jnp.where(pos < E, ev, 0.0)
        pltpu.sync_copy(ebuf, e_hbm.at[pl.ds(gbase, ECH)])

    load(0, cidxA, ridxA, g1bA, g2bA, semA)

    def body(c2, _):
        c0 = 2 * c2
        load(c0 + 1, cidxB, ridxB, g1bB, g2bB, semB)
        drain(g1bA, g2bA, semA)
        compute(c0, g1bA, g2bA)

        @pl.when(c2 < nch // 2 - 1)
        def _():
            load(c0 + 2, cidxA, ridxA, g1bA, g2bA, semA)

        drain(g1bB, g2bB, semB)
        compute(c0 + 1, g1bB, g2bB)
        return 0

    lax.fori_loop(0, nch // 2, body, 0)


def _msg_body(Nn, EPT, h5_hbm, ei3_hbm, ep_hbm, z_hbm, out_hbm,
              gidxA, sidxA, gidxB, sidxB, evA, evB, hbufA, hbufB, acc,
              semgA, semgB, semsA, semsB, semiA, semiB):
    cid = lax.axis_index("c")
    sid = lax.axis_index("s")
    nst = Nn // NSUB
    nrow = MCH // 128
    nch = EPT // MCH
    gsrc = ei3_hbm.at[1 - cid]
    ssrc = ei3_hbm.at[cid]

    def fire_idx(c, gidx, sidx, ev_, semi):
        rbase = sid * (EPT // 128) + c * nrow
        pltpu.async_copy(gsrc.at[pl.ds(rbase, nrow)], gidx, semi)
        pltpu.async_copy(ssrc.at[pl.ds(rbase, nrow)], sidx, semi)
        pltpu.async_copy(ep_hbm.at[pl.ds(sid * EPT + c * MCH, MCH)],
                         ev_, semi)

    def wait_idx(gidx, sidx, ev_, semi):
        pltpu.make_async_copy(gsrc.at[pl.ds(0, nrow)], gidx, semi).wait()
        pltpu.make_async_copy(ssrc.at[pl.ds(0, nrow)], sidx, semi).wait()
        pltpu.make_async_copy(ep_hbm.at[pl.ds(0, MCH)], ev_, semi).wait()

    def compute(ev_, hbuf):
        @plsc.parallel_loop(0, MCH // 16, unroll=2)
        def grp(g):
            evv = ev_[pl.ds(g * 16, 16)]
            for j in range(16):
                i = g * 16 + j
                ebc = jnp.full((16,), evv[j], jnp.float32)
                hbuf[i, pl.ds(0, 16)] = hbuf[i, pl.ds(0, 16)] * ebc
                hbuf[i, pl.ds(16, 16)] = hbuf[i, pl.ds(16, 16)] * ebc

    for slab in range(NSLAB):
        tab = h5_hbm.at[slab]
        dumm = out_hbm.at[cid, slab]

        def fire_g(gidx, hbuf, semg):
            for j in range(nrow):
                pltpu.async_copy(tab.at[gidx.at[j]],
                                 hbuf.at[pl.ds(j * 128, 128)], semg)

        def drain_g(hbuf, semg):
            pltpu.make_async_copy(tab.at[pl.ds(0, MCH)], hbuf, semg).wait()

        def fire_s(sidx, hbuf, sems):
            for j in range(nrow):
                pltpu.async_copy(hbuf.at[pl.ds(j * 128, 128)],
                                 acc.at[sidx.at[j]], sems, add=True)

        def drain_s(hbuf, sems):
            pltpu.make_async_copy(dumm.at[pl.ds(0, MCH)], hbuf, sems).wait()

        pltpu.sync_copy(z_hbm, acc.at[pl.ds(sid * nst, nst)])
        plsc.subcore_barrier()
        fire_idx(0, gidxA, sidxA, evA, semiA)
        wait_idx(gidxA, sidxA, evA, semiA)
        fire_g(gidxA, hbufA, semgA)
        fire_idx(1, gidxB, sidxB, evB, semiB)

        def body(c2, _):
            c0 = 2 * c2
            wait_idx(gidxB, sidxB, evB, semiB)
            fire_g(gidxB, hbufB, semgB)
            drain_g(hbufA, semgA)
            compute(evA, hbufA)
            fire_s(sidxA, hbufA, semsA)
            drain_g(hbufB, semgB)
            compute(evB, hbufB)
            drain_s(hbufA, semsA)

            @pl.when(c0 + 2 < nch)
            def _():
                fire_idx(c0 + 2, gidxA, sidxA, evA, semiA)

            fire_s(sidxB, hbufB, semsB)
            drain_s(hbufB, semsB)

            @pl.when(c0 + 3 < nch)
            def _():
                fire_idx(c0 + 3, gidxB, sidxB, evB, semiB)

            @pl.when(c0 + 2 < nch)
            def _():
                wait_idx(gidxA, sidxA, evA, semiA)
                fire_g(gidxA, hbufA, semgA)

            return 0

        lax.fori_loop(0, nch // 2, body, 0)
        plsc.subcore_barrier()
        pltpu.sync_copy(acc.at[pl.ds(sid * nst, nst)],
                        out_hbm.at[cid, slab].at[pl.ds(sid * nst, nst)])
        plsc.subcore_barrier()


def kernel(x, edge_index, W_in, b_in, We1, be1, We2, be2, Wn1, bn1, Wn2, bn2):
    N, IN_DIM = x.shape
    E = edge_index.shape[1]
    HID = W_in.shape[1]
    D = IN_DIM + HID

    row = edge_index[0]
    col = edge_index[1]

    EALIGN = 49152
    Epad = ((E + EALIGN - 1) // EALIGN) * EALIGN
    pad_idx = jnp.arange(Epad - E, dtype=jnp.int32) % N
    colp = jnp.concatenate([col, pad_idx])
    rowp = jnp.concatenate([row, pad_idx])
    cp2 = colp.reshape(-1, 128)
    rp2 = rowp.reshape(-1, 128)
    ei3 = jnp.stack([cp2, rp2])

    w = We2[:, 0]
    t16 = jnp.arange(16)
    worder = jnp.stack([j * 32 + 2 * t16 + p
                        for j in range(2) for p in range(2)])
    wtab = jnp.concatenate([
        2.0 * w[worder],
        jnp.full((1, 16), be2[0] - jnp.sum(w), jnp.float32),
    ], axis=0)
    zstripe = jnp.zeros((N // NSUB, SLABW), jnp.float32)

    We1a = We1[:D]
    We1b = We1[D:]
    be1r = be1.reshape(1, HID)
    binr = b_in.reshape(1, HID)
    bn1r = bn1.reshape(1, HID)
    bn2r = bn2.reshape(1, HID)

    BN = _pick_block(N, 2000)

    init_call = pl.pallas_call(
        _init_body,
        grid=(N // BN,),
        in_specs=[
            pl.BlockSpec((BN, IN_DIM), lambda i: (i, 0)),
            _full((IN_DIM, HID)), _full((1, HID)),
            _full((D, HID)), _full((D, HID)), _full((1, HID)),
        ],
        out_specs=[
            pl.BlockSpec((NSLAB, BN, SLABW), lambda i: (0, i, 0)),
            pl.BlockSpec((BN, HID), lambda i: (i, 0)),
            pl.BlockSpec((BN, HID), lambda i: (i, 0)),
        ],
        out_shape=[
            jax.ShapeDtypeStruct((NSLAB, N, SLABW), jnp.float32),
            jax.ShapeDtypeStruct((N, HID), jnp.bfloat16),
            jax.ShapeDtypeStruct((N, HID), jnp.bfloat16),
        ],
    )

    node_call = pl.pallas_call(
        _node_body,
        grid=(N // BN,),
        in_specs=[
            pl.BlockSpec((2, NSLAB, BN, SLABW), lambda i: (0, 0, i, 0)),
            pl.BlockSpec((NSLAB, BN, SLABW), lambda i: (0, i, 0)),
            pl.BlockSpec((BN, IN_DIM), lambda i: (i, 0)),
            _full((3 * D, HID)), _full((1, HID)),
            _full((HID, HID)), _full((1, HID)),
            _full((D, HID)), _full((D, HID)), _full((1, HID)),
        ],
        out_specs=[
            pl.BlockSpec((NSLAB, BN, SLABW), lambda i: (0, i, 0)),
            pl.BlockSpec((BN, HID), lambda i: (i, 0)),
            pl.BlockSpec((BN, HID), lambda i: (i, 0)),
        ],
        out_shape=[
            jax.ShapeDtypeStruct((NSLAB, N, SLABW), jnp.float32),
            jax.ShapeDtypeStruct((N, HID), jnp.bfloat16),
            jax.ShapeDtypeStruct((N, HID), jnp.bfloat16),
        ],
    )

    mesh = plsc.VectorSubcoreMesh(core_axis_name="c", subcore_axis_name="s")

    epass_call = pl.kernel(
        functools.partial(_epass_body, E, Epad // NW),
        out_type=jax.ShapeDtypeStruct((Epad,), jnp.float32),
        mesh=mesh,
        compiler_params=_SC_PARAMS,
        scratch_types=[
            pltpu.VMEM((ECH // 128, 128), jnp.int32),
            pltpu.VMEM((ECH // 128, 128), jnp.int32),
            pltpu.VMEM((ECH // 128, 128), jnp.int32),
            pltpu.VMEM((ECH // 128, 128), jnp.int32),
            pltpu.VMEM((ECH, HID // 2), jnp.int32),
            pltpu.VMEM((ECH, HID // 2), jnp.int32),
            pltpu.VMEM((ECH, HID // 2), jnp.int32),
            pltpu.VMEM((ECH, HID // 2), jnp.int32),
            pltpu.VMEM((ECH * 16,), jnp.float32),
            pltpu.VMEM((ECH,), jnp.float32),
            pltpu.VMEM((5, 16), jnp.float32),
            pltpu.SemaphoreType.DMA,
            pltpu.SemaphoreType.DMA,
        ],
    )

    msg_call = pl.kernel(
        functools.partial(_msg_body, N, Epad // NSUB),
        out_type=jax.ShapeDtypeStruct((2, NSLAB, N, SLABW), jnp.float32),
        mesh=mesh,
        compiler_params=_SC_PARAMS,
        scratch_types=[
            pltpu.VMEM((MCH // 128, 128), jnp.int32),
            pltpu.VMEM((MCH // 128, 128), jnp.int32),
            pltpu.VMEM((MCH // 128, 128), jnp.int32),
            pltpu.VMEM((MCH // 128, 128), jnp.int32),
            pltpu.VMEM((MCH,), jnp.float32),
            pltpu.VMEM((MCH,), jnp.float32),
            pltpu.VMEM((MCH, SLABW), jnp.float32),
            pltpu.VMEM((MCH, SLABW), jnp.float32),
            pltpu.VMEM_SHARED((N, SLABW), jnp.float32),
            pltpu.SemaphoreType.DMA,
            pltpu.SemaphoreType.DMA,
            pltpu.SemaphoreType.DMA,
            pltpu.SemaphoreType.DMA,
            pltpu.SemaphoreType.DMA,
            pltpu.SemaphoreType.DMA,
        ],
    )

    h5, G1, G2 = init_call(x, W_in, binr, We1a, We1b, be1r)

    for _ in range(3):
        ep = epass_call(
            lax.bitcast_convert_type(G1.reshape(N, HID // 2, 2), jnp.int32),
            lax.bitcast_convert_type(G2.reshape(N, HID // 2, 2), jnp.int32),
            cp2, rp2, wtab)
        m2 = msg_call(h5, ei3, ep, zstripe)
        h5, G1, G2 = node_call(m2, h5, x, Wn1, bn1r, Wn2, bn2r,
                               We1a, We1b, be1r)

    ep = epass_call(
            lax.bitcast_convert_type(G1.reshape(N, HID // 2, 2), jnp.int32),
            lax.bitcast_convert_type(G2.reshape(N, HID // 2, 2), jnp.int32),
            cp2, rp2, wtab)
    return ep[:E]


# speedup vs baseline: 1.2364x; 1.2364x over previous
import functools

import jax
import jax.numpy as jnp
from jax import lax
from jax.experimental import pallas as pl
from jax.experimental.pallas import tpu as pltpu
from jax.experimental.pallas import tpu_sc as plsc

NC = 2
NSUB = 16
NW = NC * NSUB
NSLAB = 3
SLABW = 32
ECH = 256
MCH = 384

_SC_PARAMS = pltpu.CompilerParams(
    needs_layout_passes=False, use_tc_tiling_on_sc=False)


def _pick_block(n, target):
    for b in range(target, 0, -1):
        if n % b == 0:
            return b
    return n


def _pack_h(h):
    n, d = h.shape
    hp = jnp.concatenate(
        [h, jnp.zeros((n, SLABW * NSLAB - d), jnp.float32)], axis=-1)
    return jnp.stack(
        [hp[:, SLABW * s:SLABW * (s + 1)] for s in range(NSLAB)])


def _init_body(x_ref, win_ref, bin_ref, we1a_ref, we1b_ref, be1_ref,
               h5_ref, g1_ref, g2_ref):
    X = x_ref[...]
    H = jnp.tanh(X @ win_ref[...] + bin_ref[...])
    h = jnp.concatenate([H, X], axis=-1)
    h5_ref[...] = _pack_h(h)
    g1_ref[...] = h @ we1a_ref[...]
    g2_ref[...] = h @ we1b_ref[...] + be1_ref[...]


def _unpack(m5, d):
    parts = [m5[s] for s in range(NSLAB - 1)]
    parts.append(m5[NSLAB - 1][:, :d - SLABW * (NSLAB - 1)])
    return jnp.concatenate(parts, axis=-1)


def _node_body(m2_ref, h5_ref, x_ref, wn1_ref, bn1_ref, wn2_ref,
               bn2_ref, we1a_ref, we1b_ref, be1_ref,
               h5o_ref, g1_ref, g2_ref):
    X = x_ref[...]
    d = wn1_ref.shape[0] // 3
    mi = _unpack(m2_ref[0], d)
    mo = _unpack(m2_ref[1], d)
    h = _unpack(h5_ref[...], d)
    M = jnp.concatenate([mi, mo, h], axis=-1)
    T = jnp.tanh(M @ wn1_ref[...] + bn1_ref[...])
    Hn = jnp.tanh(T @ wn2_ref[...] + bn2_ref[...])
    hn = jnp.concatenate([Hn, X], axis=-1)
    h5o_ref[...] = _pack_h(hn)
    g1_ref[...] = hn @ we1a_ref[...]
    g2_ref[...] = hn @ we1b_ref[...] + be1_ref[...]


def _full(shape):
    return pl.BlockSpec(shape, lambda i: (0,) * len(shape))


def _epass_body(E, EPT, g1_hbm, g2_hbm, cp2_hbm, rp2_hbm, wtab_hbm,
                e_hbm, cidxA, ridxA, cidxB, ridxB, g1bA, g2bA, g1bB, g2bB,
                eb2, ebuf, wbuf, semA, semB):
    cid = lax.axis_index("c")
    sid = lax.axis_index("s")
    wid = sid * NC + cid
    pltpu.sync_copy(wtab_hbm, wbuf)
    w2 = [wbuf[j] for j in range(4)]
    c0v = wbuf[4]
    iota = lax.iota(jnp.int32, 16)
    nrow = ECH // 128
    nch = EPT // ECH

    def load(c, cidx, ridx, g1b, g2b, sem):
        rbase = wid * (EPT // 128) + c * nrow
        pltpu.sync_copy(cp2_hbm.at[pl.ds(rbase, nrow)], cidx)
        pltpu.sync_copy(rp2_hbm.at[pl.ds(rbase, nrow)], ridx)
        for j in range(nrow):
            pltpu.async_copy(
                g1_hbm.at[cidx.at[j]], g1b.at[pl.ds(j * 128, 128)], sem)
            pltpu.async_copy(
                g2_hbm.at[ridx.at[j]], g2b.at[pl.ds(j * 128, 128)], sem)

    def drain(g1b, g2b, sem):
        pltpu.make_async_copy(g1_hbm.at[pl.ds(0, ECH)], g1b, sem).wait()
        pltpu.make_async_copy(g2_hbm.at[pl.ds(0, ECH)], g2b, sem).wait()

    def compute(c, g1b, g2b):
        @plsc.parallel_loop(0, ECH, unroll=4)
        def edge(i):
            acc = jnp.zeros((16,), jnp.float32)
            for j in range(4):
                v = g1b[i, pl.ds(j * 16, 16)] + g2b[i, pl.ds(j * 16, 16)]
                den = jnp.exp(v * (-2.0)) + 1.0
                acc = acc + w2[j] / den
            eb2[pl.ds(i * 16, 16)] = acc
        gbase = wid * EPT + c * ECH

        @plsc.parallel_loop(0, ECH // 16, unroll=2)
        def sig(g):
            base = g * 256 + iota * 16
            sv = jnp.zeros((16,), jnp.float32)
            for k in range(16):
                sv = sv + plsc.load_gather(eb2, [base + k])
            ev = 1.0 / (1.0 + jnp.exp(-(sv + c0v)))
            pos = gbase + g * 16 + iota
            ebuf[pl.ds(g * 16, 16)] = jnp.where(pos < E, ev, 0.0)
        pltpu.sync_copy(ebuf, e_hbm.at[pl.ds(gbase, ECH)])

    load(0, cidxA, ridxA, g1bA, g2bA, semA)

    def body(c2, _):
        c0 = 2 * c2
        load(c0 + 1, cidxB, ridxB, g1bB, g2bB, semB)
        drain(g1bA, g2bA, semA)
        compute(c0, g1bA, g2bA)

        @pl.when(c2 < nch // 2 - 1)
        def _():
            load(c0 + 2, cidxA, ridxA, g1bA, g2bA, semA)

        drain(g1bB, g2bB, semB)
        compute(c0 + 1, g1bB, g2bB)
        return 0

    lax.fori_loop(0, nch // 2, body, 0)


def _msg_body(Nn, EPT, h5_hbm, ei3_hbm, ep_hbm, z_hbm, out_hbm,
              gidxA, sidxA, gidxB, sidxB, evA, evB, hbufA, hbufB, acc,
              semgA, semgB, semsA, semsB, semiA, semiB):
    cid = lax.axis_index("c")
    sid = lax.axis_index("s")
    nst = Nn // NSUB
    nrow = MCH // 128
    nch = EPT // MCH
    gsrc = ei3_hbm.at[1 - cid]
    ssrc = ei3_hbm.at[cid]

    def fire_idx(c, gidx, sidx, ev_, semi):
        rbase = sid * (EPT // 128) + c * nrow
        pltpu.async_copy(gsrc.at[pl.ds(rbase, nrow)], gidx, semi)
        pltpu.async_copy(ssrc.at[pl.ds(rbase, nrow)], sidx, semi)
        pltpu.async_copy(ep_hbm.at[pl.ds(sid * EPT + c * MCH, MCH)],
                         ev_, semi)

    def wait_idx(gidx, sidx, ev_, semi):
        pltpu.make_async_copy(gsrc.at[pl.ds(0, nrow)], gidx, semi).wait()
        pltpu.make_async_copy(ssrc.at[pl.ds(0, nrow)], sidx, semi).wait()
        pltpu.make_async_copy(ep_hbm.at[pl.ds(0, MCH)], ev_, semi).wait()

    def compute(ev_, hbuf):
        @plsc.parallel_loop(0, MCH // 16, unroll=2)
        def grp(g):
            evv = ev_[pl.ds(g * 16, 16)]
            for j in range(16):
                i = g * 16 + j
                ebc = jnp.full((16,), evv[j], jnp.float32)
                hbuf[i, pl.ds(0, 16)] = hbuf[i, pl.ds(0, 16)] * ebc
                hbuf[i, pl.ds(16, 16)] = hbuf[i, pl.ds(16, 16)] * ebc

    for slab in range(NSLAB):
        tab = h5_hbm.at[slab]
        dumm = out_hbm.at[cid, slab]

        def fire_g(gidx, hbuf, semg):
            for j in range(nrow):
                pltpu.async_copy(tab.at[gidx.at[j]],
                                 hbuf.at[pl.ds(j * 128, 128)], semg)

        def drain_g(hbuf, semg):
            pltpu.make_async_copy(tab.at[pl.ds(0, MCH)], hbuf, semg).wait()

        def fire_s(sidx, hbuf, sems):
            for j in range(nrow):
                pltpu.async_copy(hbuf.at[pl.ds(j * 128, 128)],
                                 acc.at[sidx.at[j]], sems, add=True)

        def drain_s(hbuf, sems):
            pltpu.make_async_copy(dumm.at[pl.ds(0, MCH)], hbuf, sems).wait()

        pltpu.sync_copy(z_hbm, acc.at[pl.ds(sid * nst, nst)])
        plsc.subcore_barrier()
        fire_idx(0, gidxA, sidxA, evA, semiA)
        wait_idx(gidxA, sidxA, evA, semiA)
        fire_g(gidxA, hbufA, semgA)
        fire_idx(1, gidxB, sidxB, evB, semiB)

        def body(c2, _):
            c0 = 2 * c2
            wait_idx(gidxB, sidxB, evB, semiB)
            fire_g(gidxB, hbufB, semgB)
            drain_g(hbufA, semgA)
            compute(evA, hbufA)
            fire_s(sidxA, hbufA, semsA)
            drain_g(hbufB, semgB)
            compute(evB, hbufB)
            drain_s(hbufA, semsA)

            @pl.when(c0 + 2 < nch)
            def _():
                fire_idx(c0 + 2, gidxA, sidxA, evA, semiA)

            fire_s(sidxB, hbufB, semsB)
            drain_s(hbufB, semsB)

            @pl.when(c0 + 3 < nch)
            def _():
                fire_idx(c0 + 3, gidxB, sidxB, evB, semiB)

            @pl.when(c0 + 2 < nch)
            def _():
                wait_idx(gidxA, sidxA, evA, semiA)
                fire_g(gidxA, hbufA, semgA)

            return 0

        lax.fori_loop(0, nch // 2, body, 0)
        plsc.subcore_barrier()
        pltpu.sync_copy(acc.at[pl.ds(sid * nst, nst)],
                        out_hbm.at[cid, slab].at[pl.ds(sid * nst, nst)])
        plsc.subcore_barrier()


def kernel(x, edge_index, W_in, b_in, We1, be1, We2, be2, Wn1, bn1, Wn2, bn2):
    N, IN_DIM = x.shape
    E = edge_index.shape[1]
    HID = W_in.shape[1]
    D = IN_DIM + HID

    row = edge_index[0]
    col = edge_index[1]

    EALIGN = 49152
    Epad = ((E + EALIGN - 1) // EALIGN) * EALIGN
    pad_idx = jnp.arange(Epad - E, dtype=jnp.int32) % N
    colp = jnp.concatenate([col, pad_idx])
    rowp = jnp.concatenate([row, pad_idx])
    cp2 = colp.reshape(-1, 128)
    rp2 = rowp.reshape(-1, 128)
    ei3 = jnp.stack([cp2, rp2])

    w = We2[:, 0]
    wtab = jnp.concatenate([
        (2.0 * w).reshape(4, 16),
        jnp.full((1, 16), be2[0] - jnp.sum(w), jnp.float32),
    ], axis=0)
    zstripe = jnp.zeros((N // NSUB, SLABW), jnp.float32)

    We1a = We1[:D]
    We1b = We1[D:]
    be1r = be1.reshape(1, HID)
    binr = b_in.reshape(1, HID)
    bn1r = bn1.reshape(1, HID)
    bn2r = bn2.reshape(1, HID)

    BN = _pick_block(N, 2000)

    init_call = pl.pallas_call(
        _init_body,
        grid=(N // BN,),
        in_specs=[
            pl.BlockSpec((BN, IN_DIM), lambda i: (i, 0)),
            _full((IN_DIM, HID)), _full((1, HID)),
            _full((D, HID)), _full((D, HID)), _full((1, HID)),
        ],
        out_specs=[
            pl.BlockSpec((NSLAB, BN, SLABW), lambda i: (0, i, 0)),
            pl.BlockSpec((BN, HID), lambda i: (i, 0)),
            pl.BlockSpec((BN, HID), lambda i: (i, 0)),
        ],
        out_shape=[
            jax.ShapeDtypeStruct((NSLAB, N, SLABW), jnp.float32),
            jax.ShapeDtypeStruct((N, HID), jnp.float32),
            jax.ShapeDtypeStruct((N, HID), jnp.float32),
        ],
    )

    node_call = pl.pallas_call(
        _node_body,
        grid=(N // BN,),
        in_specs=[
            pl.BlockSpec((2, NSLAB, BN, SLABW), lambda i: (0, 0, i, 0)),
            pl.BlockSpec((NSLAB, BN, SLABW), lambda i: (0, i, 0)),
            pl.BlockSpec((BN, IN_DIM), lambda i: (i, 0)),
            _full((3 * D, HID)), _full((1, HID)),
            _full((HID, HID)), _full((1, HID)),
            _full((D, HID)), _full((D, HID)), _full((1, HID)),
        ],
        out_specs=[
            pl.BlockSpec((NSLAB, BN, SLABW), lambda i: (0, i, 0)),
            pl.BlockSpec((BN, HID), lambda i: (i, 0)),
            pl.BlockSpec((BN, HID), lambda i: (i, 0)),
        ],
        out_shape=[
            jax.ShapeDtypeStruct((NSLAB, N, SLABW), jnp.float32),
            jax.ShapeDtypeStruct((N, HID), jnp.float32),
            jax.ShapeDtypeStruct((N, HID), jnp.float32),
        ],
    )

    mesh = plsc.VectorSubcoreMesh(core_axis_name="c", subcore_axis_name="s")

    epass_call = pl.kernel(
        functools.partial(_epass_body, E, Epad // NW),
        out_type=jax.ShapeDtypeStruct((Epad,), jnp.float32),
        mesh=mesh,
        compiler_params=_SC_PARAMS,
        scratch_types=[
            pltpu.VMEM((ECH // 128, 128), jnp.int32),
            pltpu.VMEM((ECH // 128, 128), jnp.int32),
            pltpu.VMEM((ECH // 128, 128), jnp.int32),
            pltpu.VMEM((ECH // 128, 128), jnp.int32),
            pltpu.VMEM((ECH, HID), jnp.float32),
            pltpu.VMEM((ECH, HID), jnp.float32),
            pltpu.VMEM((ECH, HID), jnp.float32),
            pltpu.VMEM((ECH, HID), jnp.float32),
            pltpu.VMEM((ECH * 16,), jnp.float32),
            pltpu.VMEM((ECH,), jnp.float32),
            pltpu.VMEM((5, 16), jnp.float32),
            pltpu.SemaphoreType.DMA,
            pltpu.SemaphoreType.DMA,
        ],
    )

    msg_call = pl.kernel(
        functools.partial(_msg_body, N, Epad // NSUB),
        out_type=jax.ShapeDtypeStruct((2, NSLAB, N, SLABW), jnp.float32),
        mesh=mesh,
        compiler_params=_SC_PARAMS,
        scratch_types=[
            pltpu.VMEM((MCH // 128, 128), jnp.int32),
            pltpu.VMEM((MCH // 128, 128), jnp.int32),
            pltpu.VMEM((MCH // 128, 128), jnp.int32),
            pltpu.VMEM((MCH // 128, 128), jnp.int32),
            pltpu.VMEM((MCH,), jnp.float32),
            pltpu.VMEM((MCH,), jnp.float32),
            pltpu.VMEM((MCH, SLABW), jnp.float32),
            pltpu.VMEM((MCH, SLABW), jnp.float32),
            pltpu.VMEM_SHARED((N, SLABW), jnp.float32),
            pltpu.SemaphoreType.DMA,
            pltpu.SemaphoreType.DMA,
            pltpu.SemaphoreType.DMA,
            pltpu.SemaphoreType.DMA,
            pltpu.SemaphoreType.DMA,
            pltpu.SemaphoreType.DMA,
        ],
    )

    h5, G1, G2 = init_call(x, W_in, binr, We1a, We1b, be1r)

    for _ in range(3):
        ep = epass_call(G1, G2, cp2, rp2, wtab)
        m2 = msg_call(h5, ei3, ep, zstripe)
        h5, G1, G2 = node_call(m2, h5, x, Wn1, bn1r, Wn2, bn2r,
                               We1a, We1b, be1r)

    ep = epass_call(G1, G2, cp2, rp2, wtab)
    return ep[:E]
